# Initial kernel scaffold; baseline (speedup 1.0000x reference)
import jax
import jax.numpy as jnp
from jax.experimental import pallas as pl


def kernel(x, pos, batch, W1, b1, W2, b2):
    raise NotImplementedError("write your pallas kernel here")


# speedup vs baseline: 4.7662x; 4.7662x over previous
import jax
import jax.numpy as jnp
from jax.experimental import pallas as pl
from jax.experimental.pallas import tpu as pltpu

_NSEG = 16
_N = 32768
_B = 2048
_NB = _N // _B
_NEG = -1e30


def _fused(seg_lo_ref, seg_hi_ref,
           x_ref, pen_ref, pos_ref, W1_ref, b1_ref, W2_ref, b2_ref,
           xout_ref, posout_ref):
    i = pl.program_id(0)

    @pl.when(i == 0)
    def _init():
        xout_ref[...] = jnp.full(xout_ref.shape, -jnp.inf, jnp.float32)
        posout_ref[...] = jnp.full(posout_ref.shape, -jnp.inf, jnp.float32)

    h = jnp.maximum(
        jnp.dot(x_ref[...], W1_ref[...], preferred_element_type=jnp.float32)
        + b1_ref[...], 0.0)
    h = (jnp.dot(h, W2_ref[...], preferred_element_type=jnp.float32)
         + b2_ref[...])
    pos_blk = pos_ref[...]

    lo = seg_lo_ref[i]
    hi = seg_hi_ref[i]
    for s in range(_NSEG):
        @pl.when((lo <= s) & (s <= hi))
        def _fold(s=s):
            pen = pen_ref[:, s:s + 1]
            cand = jnp.max(h + pen, axis=0, keepdims=True)
            xout_ref[s:s + 1, :] = jnp.maximum(xout_ref[s:s + 1, :], cand)
            pcand = jnp.max(pos_blk + pen, axis=0, keepdims=True)
            posout_ref[s:s + 1, :] = jnp.maximum(posout_ref[s:s + 1, :], pcand)


def kernel(x, pos, batch, W1, b1, W2, b2):
    pen = jnp.where(
        batch[:, None] == jnp.arange(_NSEG, dtype=batch.dtype)[None, :],
        0.0, _NEG).astype(jnp.float32)
    bb = batch.reshape(_NB, _B)
    seg_lo = bb[:, 0]
    seg_hi = bb[:, -1]

    grid_spec = pltpu.PrefetchScalarGridSpec(
        num_scalar_prefetch=2,
        grid=(_NB,),
        in_specs=[
            pl.BlockSpec((_B, 128), lambda i, lo, hi: (i, 0)),
            pl.BlockSpec((_B, _NSEG), lambda i, lo, hi: (i, 0)),
            pl.BlockSpec((_B, 3), lambda i, lo, hi: (i, 0)),
            pl.BlockSpec((128, 128), lambda i, lo, hi: (0, 0)),
            pl.BlockSpec((1, 128), lambda i, lo, hi: (0, 0)),
            pl.BlockSpec((128, 256), lambda i, lo, hi: (0, 0)),
            pl.BlockSpec((1, 256), lambda i, lo, hi: (0, 0)),
        ],
        out_specs=[
            pl.BlockSpec((_NSEG, 256), lambda i, lo, hi: (0, 0)),
            pl.BlockSpec((_NSEG, 3), lambda i, lo, hi: (0, 0)),
        ],
    )
    x_out, pos_out = pl.pallas_call(
        _fused,
        grid_spec=grid_spec,
        out_shape=[
            jax.ShapeDtypeStruct((_NSEG, 256), jnp.float32),
            jax.ShapeDtypeStruct((_NSEG, 3), jnp.float32),
        ],
        compiler_params=pltpu.CompilerParams(
            dimension_semantics=("arbitrary",)),
    )(seg_lo, seg_hi, x, pen, pos, W1, b1.reshape(1, 128), W2,
      b2.reshape(1, 256))
    batch_out = jnp.arange(_NSEG, dtype=jnp.int32)
    return (x_out, pos_out, batch_out)
